# baseline (speedup 1.0000x reference)
import functools

import jax
import jax.numpy as jnp
from jax.experimental import pallas as pl
from jax.experimental.pallas import tpu as pltpu


def _body(facs_ref, x_ref, noise_hbm, o_hbm, nbuf, obuf, in_sem, out_sem,
          *, w, cutn, chunk):
    c_dim, _, half, _ = x_ref.shape
    s = pl.program_id(0)
    row0 = s * half
    n_chunks = cutn // chunk

    def in_copy(slot, k):
        return pltpu.make_async_copy(
            noise_hbm.at[pl.ds(k * chunk, chunk), :, pl.ds(row0, half), :],
            nbuf.at[slot], in_sem.at[slot])

    def out_copy(slot, k):
        return pltpu.make_async_copy(
            obuf.at[slot],
            o_hbm.at[pl.ds(k * chunk, chunk), :, pl.ds(row0, half), :],
            out_sem.at[slot])

    in_copy(0, 0).start()
    if n_chunks > 1:
        in_copy(1, 1).start()

    rows = c_dim * half
    v = x_ref[...].astype(jnp.float32).reshape(rows, 2 * w)
    top = v[:, 0:w]
    bot = v[:, w:2 * w]
    rs = top + bot
    rm = jnp.maximum(top, bot)
    i = jax.lax.broadcasted_iota(jnp.int32, (w, w // 2), 0)
    jj = jax.lax.broadcasted_iota(jnp.int32, (w, w // 2), 1)
    e0 = (i == 2 * jj).astype(jnp.float32)
    e1 = (i == 2 * jj + 1).astype(jnp.float32)

    def dot(a, b):
        return jax.lax.dot_general(
            a, b, (((1,), (0,)), ((), ())),
            preferred_element_type=jnp.float32)

    def sel_dot(a, b):
        hi = a.astype(jnp.bfloat16).astype(jnp.float32)
        lo = a - hi
        return dot(hi, b) + dot(lo, b)

    cs_ = sel_dot(rs, e0 + e1)
    cm = jnp.maximum(sel_dot(rm, e0), sel_dot(rm, e1))
    pooled = ((cs_ * 0.25 + cm) * 0.5).reshape(c_dim, half, w // 2)

    for k in range(n_chunks):
        slot = k % 2
        in_copy(slot, k).wait()
        if k >= 2:
            out_copy(slot, k - 2).wait()
        for b in range(chunk):
            fac = facs_ref[k * chunk + b]
            obuf[slot, b] = (pooled + fac * nbuf[slot, b].astype(
                jnp.float32)).astype(obuf.dtype)
        out_copy(slot, k).start()
        if k + 2 < n_chunks:
            in_copy(slot, k + 2).start()

    for k in range(max(0, n_chunks - 2), n_chunks):
        out_copy(k % 2, k).wait()


def kernel(x, facs, noise):
    N, C, H, W = x.shape
    cutn, _, cs, _ = noise.shape
    half = cs // 2
    x4 = x[0].reshape(C, 2, half, 2 * W)

    G = min(8, cutn)
    out = pl.pallas_call(
        functools.partial(_body, w=W, cutn=cutn, chunk=G),
        out_shape=jax.ShapeDtypeStruct((cutn, C, cs, cs), x.dtype),
        grid=(2,),
        in_specs=[
            pl.BlockSpec(memory_space=pltpu.MemorySpace.SMEM),
            pl.BlockSpec((C, 1, half, 2 * W), lambda h: (0, h, 0, 0)),
            pl.BlockSpec(memory_space=pltpu.MemorySpace.HBM),
        ],
        out_specs=pl.BlockSpec(memory_space=pltpu.MemorySpace.HBM),
        scratch_shapes=[
            pltpu.VMEM((2, G, C, half, cs), jnp.float32),
            pltpu.VMEM((2, G, C, half, cs), jnp.float32),
            pltpu.SemaphoreType.DMA((2,)),
            pltpu.SemaphoreType.DMA((2,)),
        ],
        compiler_params=pltpu.CompilerParams(
            dimension_semantics=("parallel",),
            vmem_limit_bytes=64 * 1024 * 1024,
        ),
    )(facs, x4, noise)

    return out


# speedup vs baseline: 1.2109x; 1.2109x over previous
import functools

import jax
import jax.numpy as jnp
from jax.experimental import pallas as pl
from jax.experimental.pallas import tpu as pltpu


def _body(facs_ref, x_ref, noise_ref, o_ref, *, w, cutn):
    c_dim, _, half, _ = x_ref.shape
    rows = c_dim * half
    v = x_ref[...].astype(jnp.float32).reshape(rows, 2 * w)
    top = v[:, 0:w]
    bot = v[:, w:2 * w]
    rs = top + bot
    rm = jnp.maximum(top, bot)
    i = jax.lax.broadcasted_iota(jnp.int32, (w, w // 2), 0)
    jj = jax.lax.broadcasted_iota(jnp.int32, (w, w // 2), 1)
    e0 = (i == 2 * jj).astype(jnp.float32)
    e1 = (i == 2 * jj + 1).astype(jnp.float32)

    def dot(a, b):
        return jax.lax.dot_general(
            a, b, (((1,), (0,)), ((), ())),
            preferred_element_type=jnp.float32)

    def sel_dot(a, b):
        hi = a.astype(jnp.bfloat16).astype(jnp.float32)
        lo = a - hi
        return dot(hi, b) + dot(lo, b)

    cs_ = sel_dot(rs, e0 + e1)
    cm = jnp.maximum(sel_dot(rm, e0), sel_dot(rm, e1))
    pooled = ((cs_ * 0.25 + cm) * 0.5).reshape(c_dim, half, w // 2)

    for b in range(cutn):
        fac = facs_ref[b]
        o_ref[b] = (pooled + fac * noise_ref[b].astype(jnp.float32)).astype(
            o_ref.dtype)


def kernel(x, facs, noise):
    N, C, H, W = x.shape
    cutn, _, cs, _ = noise.shape
    half = cs // 2
    x4 = x[0].reshape(C, 2, half, 2 * W)

    out = pl.pallas_call(
        functools.partial(_body, w=W, cutn=cutn),
        out_shape=jax.ShapeDtypeStruct((cutn, C, cs, cs), x.dtype),
        grid=(2,),
        in_specs=[
            pl.BlockSpec(memory_space=pltpu.MemorySpace.SMEM),
            pl.BlockSpec((C, 1, half, 2 * W), lambda h: (0, h, 0, 0)),
            pl.BlockSpec((cutn, C, half, cs), lambda h: (0, 0, h, 0)),
        ],
        out_specs=pl.BlockSpec((cutn, C, half, cs), lambda h: (0, 0, h, 0)),
        compiler_params=pltpu.CompilerParams(
            dimension_semantics=("parallel",),
            vmem_limit_bytes=64 * 1024 * 1024,
        ),
    )(facs, x4, noise)

    return out
